# baseline (speedup 1.0000x reference)
import functools

import jax
import jax.numpy as jnp
from jax import lax
from jax.experimental import pallas as pl
from jax.experimental.pallas import tpu as pltpu
from jax.experimental.pallas import tpu_sc as plsc

ND = 10000
D = 128
DIN = 256
E = 131072
NC = 2
NS = 16
CH = 128
NBUF = 8
EROWS = E // CH
RPT = EROWS // (NC * NS)
NPT = ND // NS


def _split3(r, oa_ref, ob_ref, os_ref):
    oa_ref[...] = r[:, :D].astype(jnp.bfloat16)
    ob_ref[...] = r[:, D:2 * D].astype(jnp.bfloat16)
    os_ref[...] = r[:, 2 * D:]


def _projmsg_body(x_ref, wp_ref, bp_ref, w_ref, h_ref, oa_ref, ob_ref,
                  os_ref):
    h = jnp.maximum(
        jnp.dot(x_ref[...], wp_ref[...], preferred_element_type=jnp.float32)
        + bp_ref[...], 0.0)
    h_ref[...] = h
    _split3(jnp.dot(h, w_ref[...], preferred_element_type=jnp.float32),
            oa_ref, ob_ref, os_ref)


def _projmsg(x, wp_t, bp, w_cat):
    blk = 2000
    return pl.pallas_call(
        _projmsg_body,
        grid=(ND // blk,),
        in_specs=[pl.BlockSpec((blk, DIN), lambda i: (i, 0)),
                  pl.BlockSpec((DIN, D), lambda i: (0, 0)),
                  pl.BlockSpec((1, D), lambda i: (0, 0)),
                  pl.BlockSpec((D, 3 * D), lambda i: (0, 0))],
        out_specs=[pl.BlockSpec((blk, D), lambda i: (i, 0))] * 4,
        out_shape=[jax.ShapeDtypeStruct((ND, D), jnp.float32),
                   jax.ShapeDtypeStruct((ND, D), jnp.bfloat16),
                   jax.ShapeDtypeStruct((ND, D), jnp.bfloat16),
                   jax.ShapeDtypeStruct((ND, D), jnp.float32)] ,
    )(x, wp_t, bp, w_cat)


def _combine_val(h_ref, s_ref, agg_ref, cnt_ref, g_ref, b_ref):
    agg = (agg_ref[0].astype(jnp.float32) + agg_ref[1].astype(jnp.float32))
    cnt = jnp.sum(cnt_ref[...], axis=1, keepdims=True)
    t = jnp.maximum(s_ref[...] + agg / jnp.maximum(cnt, 1.0), 0.0) + h_ref[...]
    mu = jnp.mean(t, axis=-1, keepdims=True)
    var = jnp.mean((t - mu) ** 2, axis=-1, keepdims=True)
    return (t - mu) * lax.rsqrt(var + 1e-5) * g_ref[...] + b_ref[...]


def _combine_body(h_ref, s_ref, agg_ref, cnt_ref, g_ref, b_ref, o_ref):
    o_ref[...] = _combine_val(h_ref, s_ref, agg_ref, cnt_ref, g_ref, b_ref)


def _combmsg_body(h_ref, s_ref, agg_ref, cnt_ref, g_ref, b_ref, w_ref,
                  hn_ref, oa_ref, ob_ref, os_ref):
    y = _combine_val(h_ref, s_ref, agg_ref, cnt_ref, g_ref, b_ref)
    hn_ref[...] = y
    _split3(jnp.dot(y, w_ref[...], preferred_element_type=jnp.float32),
            oa_ref, ob_ref, os_ref)


def _combmsg(h, sf, agg, cnt, g, b, w_cat):
    blk = 2000
    return pl.pallas_call(
        _combmsg_body,
        grid=(ND // blk,),
        in_specs=[pl.BlockSpec((blk, D), lambda i: (i, 0)),
                  pl.BlockSpec((blk, D), lambda i: (i, 0)),
                  pl.BlockSpec((2, blk, D), lambda i: (0, i, 0)),
                  pl.BlockSpec((blk, NC * NS), lambda i: (i, 0)),
                  pl.BlockSpec((1, D), lambda i: (0, 0)),
                  pl.BlockSpec((1, D), lambda i: (0, 0)),
                  pl.BlockSpec((D, 3 * D), lambda i: (0, 0))],
        out_specs=[pl.BlockSpec((blk, D), lambda i: (i, 0))] * 4,
        out_shape=[jax.ShapeDtypeStruct((ND, D), jnp.float32),
                   jax.ShapeDtypeStruct((ND, D), jnp.bfloat16),
                   jax.ShapeDtypeStruct((ND, D), jnp.bfloat16),
                   jax.ShapeDtypeStruct((ND, D), jnp.float32)] ,
    )(h, sf, agg, cnt, g, b, w_cat)


def _combine(h, s, agg, cnt, g, b):
    blk = 2000
    return pl.pallas_call(
        _combine_body,
        grid=(ND // blk,),
        in_specs=[pl.BlockSpec((blk, D), lambda i: (i, 0)),
                  pl.BlockSpec((blk, D), lambda i: (i, 0)),
                  pl.BlockSpec((2, blk, D), lambda i: (0, i, 0)),
                  pl.BlockSpec((blk, NC * NS), lambda i: (i, 0)),
                  pl.BlockSpec((1, D), lambda i: (0, 0)),
                  pl.BlockSpec((1, D), lambda i: (0, 0))],
        out_specs=pl.BlockSpec((blk, D), lambda i: (i, 0)),
        out_shape=jax.ShapeDtypeStruct((ND, D), jnp.float32),
    )(h, s, agg, cnt, g, b)


def _make_sc_phase(with_cnt):
    def body(*refs):
        if with_cnt:
            (m_a, m_b, s_a, d_a, s_b, d_b, zrows, agg, cnt,
             acc, src_v, dst_v, *rest) = refs
        else:
            (m_a, m_b, s_a, d_a, s_b, d_b, zrows, agg,
             acc, src_v, dst_v, *rest) = refs
        rows = tuple(rest[:NBUF])
        if with_cnt:
            cnt_loc = rest[NBUF]
            sems = rest[NBUF + 1:]
        else:
            sems = rest[NBUF:]
        gsem = tuple(sems[:NBUF])
        ssem = tuple(sems[NBUF:])
        cid = lax.axis_index("c")
        sid = lax.axis_index("s")

        zero16 = jnp.zeros((16,), jnp.float32)
        one16 = jnp.ones((16,), jnp.float32)

        if with_cnt:
            def _zcnt(i, carry):
                cnt_loc[pl.ds(i * 16, 16)] = zero16
                return carry
            lax.fori_loop(0, ND // 16, _zcnt, 0)

        def _z(k, carry):
            pltpu.sync_copy(zrows, acc.at[pl.ds(sid * NPT + k * 125, 125)])
            return carry
        lax.fori_loop(0, NPT // 125, _z, 0)
        plsc.subcore_barrier()

        def do_rel(msg, s2d, d2d):
            row0 = cid * (EROWS // 2) + sid * RPT
            pltpu.sync_copy(s2d.at[pl.ds(row0, RPT)], src_v)
            pltpu.sync_copy(d2d.at[pl.ds(row0, RPT)], dst_v)

            for b in range(NBUF):
                pltpu.async_copy(msg.at[src_v.at[b]], rows[b], gsem[b])

            def _grp(o, carry):
                c0 = NBUF * o
                sdesc = []
                for b in range(NBUF):
                    c = c0 + b
                    pltpu.make_async_copy(
                        msg.at[src_v.at[c]], rows[b], gsem[b]).wait()
                    sdesc.append(pltpu.async_copy(
                        rows[b], acc.at[dst_v.at[c]], ssem[b], add=True))
                    if with_cnt:
                        for j in range(CH // 16):
                            idx = dst_v[c, pl.ds(j * 16, 16)]
                            plsc.addupdate_scatter(cnt_loc, [idx], one16)
                for b in range(NBUF):
                    sdesc[b].wait()

                    @pl.when(o < RPT // NBUF - 1)
                    def _():
                        pltpu.async_copy(
                            msg.at[src_v.at[c0 + b + NBUF]], rows[b], gsem[b])
                return carry
            lax.fori_loop(0, RPT // NBUF, _grp, 0)

        do_rel(m_a, s_a, d_a)
        do_rel(m_b, s_b, d_b)

        plsc.subcore_barrier()
        pltpu.sync_copy(acc.at[pl.ds(sid * NPT, NPT)],
                        agg.at[cid].at[pl.ds(sid * NPT, NPT)])
        if with_cnt:
            pltpu.sync_copy(cnt_loc, cnt.at[cid * NS + sid])

    out_type = [jax.ShapeDtypeStruct((NC, ND, D), jnp.bfloat16)]
    scratch = [
        pltpu.VMEM_SHARED((ND, D), jnp.bfloat16),
        pltpu.VMEM((RPT, CH), jnp.int32),
        pltpu.VMEM((RPT, CH), jnp.int32),
    ] + [pltpu.VMEM((CH, D), jnp.bfloat16)] * NBUF
    if with_cnt:
        out_type.append(jax.ShapeDtypeStruct((NC * NS, ND), jnp.float32))
        scratch.append(pltpu.VMEM((ND,), jnp.float32))
    scratch += [pltpu.SemaphoreType.DMA] * (2 * NBUF)
    return pl.kernel(
        body,
        out_type=out_type,
        mesh=plsc.VectorSubcoreMesh(core_axis_name="c", subcore_axis_name="s",
                                    num_cores=NC, num_subcores=NS),
        compiler_params=pltpu.CompilerParams(use_tc_tiling_on_sc=False,
                                             needs_layout_passes=False),
        scratch_types=scratch,
    )


_sc_phase_cnt = _make_sc_phase(True)
_sc_phase_nocnt = _make_sc_phase(False)


def kernel(x_drug, x_protein, edge_dd, edge_dt, edge_td, edge_pp,
           Wp_drug, bp_drug, Wp_protein, bp_protein,
           W_rel, W_self, ln_gamma, ln_beta):
    def split(e):
        e = e.astype(jnp.int32)
        return e[0].reshape(EROWS, CH), e[1].reshape(EROWS, CH)

    s_dd, d_dd = split(edge_dd)
    s_dt, d_dt = split(edge_dt)
    s_td, d_td = split(edge_td)
    s_pp, d_pp = split(edge_pp)
    zrows = jnp.zeros((125, D), jnp.bfloat16)

    w_d0 = jnp.concatenate(
        [W_rel[0, 0].T, W_rel[0, 1].T, W_self[0, 0].T], axis=1)
    w_p0 = jnp.concatenate(
        [W_rel[0, 2].T, W_rel[0, 3].T, W_self[0, 1].T], axis=1)
    w_d1 = jnp.concatenate(
        [W_rel[1, 0].T, W_rel[1, 1].T, W_self[1, 0].T], axis=1)
    w_p1 = jnp.concatenate(
        [W_rel[1, 2].T, W_rel[1, 3].T, W_self[1, 1].T], axis=1)

    h_d, m_dd, m_dt, self_d = _projmsg(x_drug, Wp_drug.T, bp_drug[None, :],
                                       w_d0)
    h_p, m_td, m_pp, self_p = _projmsg(x_protein, Wp_protein.T,
                                       bp_protein[None, :], w_p0)
    aggd, cntd = _sc_phase_cnt(m_dd, m_td, s_dd, d_dd, s_td, d_td, zrows)
    cntd_t = cntd.T
    h_d, m_dd1, m_dt1, self_d1 = _combmsg(
        h_d, self_d, aggd, cntd_t,
        ln_gamma[0, 0][None, :], ln_beta[0, 0][None, :], w_d1)
    aggp, cntp = _sc_phase_cnt(m_dt, m_pp, s_dt, d_dt, s_pp, d_pp, zrows)
    cntp_t = cntp.T
    h_p, m_td1, m_pp1, self_p1 = _combmsg(
        h_p, self_p, aggp, cntp_t,
        ln_gamma[0, 1][None, :], ln_beta[0, 1][None, :], w_p1)
    aggd = _sc_phase_nocnt(m_dd1, m_td1, s_dd, d_dd, s_td, d_td, zrows)[0]
    h_d = _combine(h_d, self_d1, aggd, cntd_t,
                   ln_gamma[1, 0][None, :], ln_beta[1, 0][None, :])
    aggp = _sc_phase_nocnt(m_dt1, m_pp1, s_dt, d_dt, s_pp, d_pp, zrows)[0]
    h_p = _combine(h_p, self_p1, aggp, cntp_t,
                   ln_gamma[1, 1][None, :], ln_beta[1, 1][None, :])

    return jnp.concatenate([h_d, h_p], axis=0)


# speedup vs baseline: 1.0056x; 1.0056x over previous
import functools

import jax
import jax.numpy as jnp
from jax import lax
from jax.experimental import pallas as pl
from jax.experimental.pallas import tpu as pltpu
from jax.experimental.pallas import tpu_sc as plsc

ND = 10000
D = 128
DIN = 256
E = 131072
NC = 2
NS = 16
CH = 64
NBUF = 8
EROWS = E // CH
RPT = EROWS // (NC * NS)
NPT = ND // NS


def _split3(r, oa_ref, ob_ref, os_ref):
    oa_ref[...] = r[:, :D].astype(jnp.bfloat16)
    ob_ref[...] = r[:, D:2 * D].astype(jnp.bfloat16)
    os_ref[...] = r[:, 2 * D:]


def _projmsg_body(x_ref, wp_ref, bp_ref, w_ref, h_ref, oa_ref, ob_ref,
                  os_ref):
    h = jnp.maximum(
        jnp.dot(x_ref[...], wp_ref[...], preferred_element_type=jnp.float32)
        + bp_ref[...], 0.0)
    h_ref[...] = h
    _split3(jnp.dot(h, w_ref[...], preferred_element_type=jnp.float32),
            oa_ref, ob_ref, os_ref)


def _projmsg(x, wp_t, bp, w_cat):
    blk = 2000
    return pl.pallas_call(
        _projmsg_body,
        grid=(ND // blk,),
        in_specs=[pl.BlockSpec((blk, DIN), lambda i: (i, 0)),
                  pl.BlockSpec((DIN, D), lambda i: (0, 0)),
                  pl.BlockSpec((1, D), lambda i: (0, 0)),
                  pl.BlockSpec((D, 3 * D), lambda i: (0, 0))],
        out_specs=[pl.BlockSpec((blk, D), lambda i: (i, 0))] * 4,
        out_shape=[jax.ShapeDtypeStruct((ND, D), jnp.float32),
                   jax.ShapeDtypeStruct((ND, D), jnp.bfloat16),
                   jax.ShapeDtypeStruct((ND, D), jnp.bfloat16),
                   jax.ShapeDtypeStruct((ND, D), jnp.float32)] ,
    )(x, wp_t, bp, w_cat)


def _combine_val(h_ref, s_ref, agg_ref, cnt_ref, g_ref, b_ref):
    agg = (agg_ref[0].astype(jnp.float32) + agg_ref[1].astype(jnp.float32))
    cnt = jnp.sum(cnt_ref[...], axis=1, keepdims=True)
    t = jnp.maximum(s_ref[...] + agg / jnp.maximum(cnt, 1.0), 0.0) + h_ref[...]
    mu = jnp.mean(t, axis=-1, keepdims=True)
    var = jnp.mean((t - mu) ** 2, axis=-1, keepdims=True)
    return (t - mu) * lax.rsqrt(var + 1e-5) * g_ref[...] + b_ref[...]


def _combine_body(h_ref, s_ref, agg_ref, cnt_ref, g_ref, b_ref, o_ref):
    o_ref[...] = _combine_val(h_ref, s_ref, agg_ref, cnt_ref, g_ref, b_ref)


def _combmsg_body(h_ref, s_ref, agg_ref, cnt_ref, g_ref, b_ref, w_ref,
                  hn_ref, oa_ref, ob_ref, os_ref):
    y = _combine_val(h_ref, s_ref, agg_ref, cnt_ref, g_ref, b_ref)
    hn_ref[...] = y
    _split3(jnp.dot(y, w_ref[...], preferred_element_type=jnp.float32),
            oa_ref, ob_ref, os_ref)


def _combmsg(h, sf, agg, cnt, g, b, w_cat):
    blk = 2000
    return pl.pallas_call(
        _combmsg_body,
        grid=(ND // blk,),
        in_specs=[pl.BlockSpec((blk, D), lambda i: (i, 0)),
                  pl.BlockSpec((blk, D), lambda i: (i, 0)),
                  pl.BlockSpec((2, blk, D), lambda i: (0, i, 0)),
                  pl.BlockSpec((blk, NC * NS), lambda i: (i, 0)),
                  pl.BlockSpec((1, D), lambda i: (0, 0)),
                  pl.BlockSpec((1, D), lambda i: (0, 0)),
                  pl.BlockSpec((D, 3 * D), lambda i: (0, 0))],
        out_specs=[pl.BlockSpec((blk, D), lambda i: (i, 0))] * 4,
        out_shape=[jax.ShapeDtypeStruct((ND, D), jnp.float32),
                   jax.ShapeDtypeStruct((ND, D), jnp.bfloat16),
                   jax.ShapeDtypeStruct((ND, D), jnp.bfloat16),
                   jax.ShapeDtypeStruct((ND, D), jnp.float32)] ,
    )(h, sf, agg, cnt, g, b, w_cat)


def _combine(h, s, agg, cnt, g, b):
    blk = 2000
    return pl.pallas_call(
        _combine_body,
        grid=(ND // blk,),
        in_specs=[pl.BlockSpec((blk, D), lambda i: (i, 0)),
                  pl.BlockSpec((blk, D), lambda i: (i, 0)),
                  pl.BlockSpec((2, blk, D), lambda i: (0, i, 0)),
                  pl.BlockSpec((blk, NC * NS), lambda i: (i, 0)),
                  pl.BlockSpec((1, D), lambda i: (0, 0)),
                  pl.BlockSpec((1, D), lambda i: (0, 0))],
        out_specs=pl.BlockSpec((blk, D), lambda i: (i, 0)),
        out_shape=jax.ShapeDtypeStruct((ND, D), jnp.float32),
    )(h, s, agg, cnt, g, b)


def _make_sc_phase(with_cnt):
    def body(*refs):
        if with_cnt:
            (m_a, m_b, s_a, d_a, s_b, d_b, zrows, agg, cnt,
             acc, src_v, dst_v, *rest) = refs
        else:
            (m_a, m_b, s_a, d_a, s_b, d_b, zrows, agg,
             acc, src_v, dst_v, *rest) = refs
        rows = tuple(rest[:NBUF])
        if with_cnt:
            cnt_loc = rest[NBUF]
            sems = rest[NBUF + 1:]
        else:
            sems = rest[NBUF:]
        gsem = tuple(sems[:NBUF])
        ssem = tuple(sems[NBUF:])
        cid = lax.axis_index("c")
        sid = lax.axis_index("s")

        zero16 = jnp.zeros((16,), jnp.float32)
        one16 = jnp.ones((16,), jnp.float32)

        if with_cnt:
            def _zcnt(i, carry):
                cnt_loc[pl.ds(i * 16, 16)] = zero16
                return carry
            lax.fori_loop(0, ND // 16, _zcnt, 0)

        def _z(k, carry):
            pltpu.sync_copy(zrows, acc.at[pl.ds(sid * NPT + k * 125, 125)])
            return carry
        lax.fori_loop(0, NPT // 125, _z, 0)
        plsc.subcore_barrier()

        def do_rel(msg, s2d, d2d):
            row0 = cid * (EROWS // 2) + sid * RPT
            pltpu.sync_copy(s2d.at[pl.ds(row0, RPT)], src_v)
            pltpu.sync_copy(d2d.at[pl.ds(row0, RPT)], dst_v)

            for b in range(NBUF):
                pltpu.async_copy(msg.at[src_v.at[b]], rows[b], gsem[b])

            def _grp(o, carry):
                c0 = NBUF * o
                sdesc = []
                for b in range(NBUF):
                    c = c0 + b
                    pltpu.make_async_copy(
                        msg.at[src_v.at[c]], rows[b], gsem[b]).wait()
                    sdesc.append(pltpu.async_copy(
                        rows[b], acc.at[dst_v.at[c]], ssem[b], add=True))
                    if with_cnt:
                        for j in range(CH // 16):
                            idx = dst_v[c, pl.ds(j * 16, 16)]
                            plsc.addupdate_scatter(cnt_loc, [idx], one16)
                for b in range(NBUF):
                    sdesc[b].wait()

                    @pl.when(o < RPT // NBUF - 1)
                    def _():
                        pltpu.async_copy(
                            msg.at[src_v.at[c0 + b + NBUF]], rows[b], gsem[b])
                return carry
            lax.fori_loop(0, RPT // NBUF, _grp, 0)

        do_rel(m_a, s_a, d_a)
        do_rel(m_b, s_b, d_b)

        plsc.subcore_barrier()
        pltpu.sync_copy(acc.at[pl.ds(sid * NPT, NPT)],
                        agg.at[cid].at[pl.ds(sid * NPT, NPT)])
        if with_cnt:
            pltpu.sync_copy(cnt_loc, cnt.at[cid * NS + sid])

    out_type = [jax.ShapeDtypeStruct((NC, ND, D), jnp.bfloat16)]
    scratch = [
        pltpu.VMEM_SHARED((ND, D), jnp.bfloat16),
        pltpu.VMEM((RPT, CH), jnp.int32),
        pltpu.VMEM((RPT, CH), jnp.int32),
    ] + [pltpu.VMEM((CH, D), jnp.bfloat16)] * NBUF
    if with_cnt:
        out_type.append(jax.ShapeDtypeStruct((NC * NS, ND), jnp.float32))
        scratch.append(pltpu.VMEM((ND,), jnp.float32))
    scratch += [pltpu.SemaphoreType.DMA] * (2 * NBUF)
    return pl.kernel(
        body,
        out_type=out_type,
        mesh=plsc.VectorSubcoreMesh(core_axis_name="c", subcore_axis_name="s",
                                    num_cores=NC, num_subcores=NS),
        compiler_params=pltpu.CompilerParams(use_tc_tiling_on_sc=False,
                                             needs_layout_passes=False),
        scratch_types=scratch,
    )


_sc_phase_cnt = _make_sc_phase(True)
_sc_phase_nocnt = _make_sc_phase(False)


def kernel(x_drug, x_protein, edge_dd, edge_dt, edge_td, edge_pp,
           Wp_drug, bp_drug, Wp_protein, bp_protein,
           W_rel, W_self, ln_gamma, ln_beta):
    def split(e):
        e = e.astype(jnp.int32)
        return e[0].reshape(EROWS, CH), e[1].reshape(EROWS, CH)

    s_dd, d_dd = split(edge_dd)
    s_dt, d_dt = split(edge_dt)
    s_td, d_td = split(edge_td)
    s_pp, d_pp = split(edge_pp)
    zrows = jnp.zeros((125, D), jnp.bfloat16)

    w_d0 = jnp.concatenate(
        [W_rel[0, 0].T, W_rel[0, 1].T, W_self[0, 0].T], axis=1)
    w_p0 = jnp.concatenate(
        [W_rel[0, 2].T, W_rel[0, 3].T, W_self[0, 1].T], axis=1)
    w_d1 = jnp.concatenate(
        [W_rel[1, 0].T, W_rel[1, 1].T, W_self[1, 0].T], axis=1)
    w_p1 = jnp.concatenate(
        [W_rel[1, 2].T, W_rel[1, 3].T, W_self[1, 1].T], axis=1)

    h_d, m_dd, m_dt, self_d = _projmsg(x_drug, Wp_drug.T, bp_drug[None, :],
                                       w_d0)
    h_p, m_td, m_pp, self_p = _projmsg(x_protein, Wp_protein.T,
                                       bp_protein[None, :], w_p0)
    aggd, cntd = _sc_phase_cnt(m_dd, m_td, s_dd, d_dd, s_td, d_td, zrows)
    cntd_t = cntd.T
    h_d, m_dd1, m_dt1, self_d1 = _combmsg(
        h_d, self_d, aggd, cntd_t,
        ln_gamma[0, 0][None, :], ln_beta[0, 0][None, :], w_d1)
    aggp, cntp = _sc_phase_cnt(m_dt, m_pp, s_dt, d_dt, s_pp, d_pp, zrows)
    cntp_t = cntp.T
    h_p, m_td1, m_pp1, self_p1 = _combmsg(
        h_p, self_p, aggp, cntp_t,
        ln_gamma[0, 1][None, :], ln_beta[0, 1][None, :], w_p1)
    aggd = _sc_phase_nocnt(m_dd1, m_td1, s_dd, d_dd, s_td, d_td, zrows)[0]
    h_d = _combine(h_d, self_d1, aggd, cntd_t,
                   ln_gamma[1, 0][None, :], ln_beta[1, 0][None, :])
    aggp = _sc_phase_nocnt(m_dt1, m_pp1, s_dt, d_dt, s_pp, d_pp, zrows)[0]
    h_p = _combine(h_p, self_p1, aggp, cntp_t,
                   ln_gamma[1, 1][None, :], ln_beta[1, 1][None, :])

    return jnp.concatenate([h_d, h_p], axis=0)
